# baseline (speedup 1.0000x reference)
import functools

import jax
import jax.numpy as jnp
from jax import lax
from jax.experimental import pallas as pl
from jax.experimental.pallas import tpu as pltpu
from jax.experimental.pallas import tpu_sc as plsc

VOCAB = 100000
EMB = 128
SEQ = 50
BATCH = 4096
NUM_LABELS = 1024
K = SEQ * EMB

_info = plsc.get_sparse_core_info()
_NC, _NS = _info.num_cores, _info.num_subcores
_NW = _NC * _NS

_H = 4
_BATCH_H = BATCH // _H
_NROWS_H = _BATCH_H * SEQ
_PER_W = _NROWS_H // _NW
_CHUNK = 160
_NCHUNK = _PER_W // _CHUNK


def _make_gather(nrows, per_w, chunk, nchunk):
    mesh = plsc.VectorSubcoreMesh(core_axis_name="c", subcore_axis_name="s")

    @functools.partial(
        pl.kernel,
        mesh=mesh,
        out_type=jax.ShapeDtypeStruct((nrows, EMB), jnp.float32),
        scratch_types=[
            pltpu.VMEM((chunk,), jnp.int32),
            pltpu.VMEM((chunk,), jnp.int32),
            pltpu.VMEM((chunk,), jnp.int32),
            pltpu.VMEM((chunk, EMB), jnp.float32),
            pltpu.VMEM((chunk, EMB), jnp.float32),
            pltpu.VMEM((chunk, EMB), jnp.float32),
            pltpu.SemaphoreType.DMA,
            pltpu.SemaphoreType.DMA,
            pltpu.SemaphoreType.DMA,
            pltpu.SemaphoreType.DMA,
            pltpu.SemaphoreType.DMA,
            pltpu.SemaphoreType.DMA,
            pltpu.SemaphoreType.DMA,
            pltpu.SemaphoreType.DMA,
            pltpu.SemaphoreType.DMA,
        ],
    )
    def gather_k(idx_hbm, table_hbm, out_hbm,
                 ib0, ib1, ib2, rb0, rb1, rb2,
                 is0, is1, is2, gs0, gs1, gs2, ws0, ws1, ws2):
        wid = lax.axis_index("s") * _NC + lax.axis_index("c")
        base = wid * per_w
        ib = (ib0, ib1, ib2)
        rb = (rb0, rb1, rb2)
        isem = (is0, is1, is2)
        gsem = (gs0, gs1, gs2)
        wsem = (ws0, ws1, ws2)

        def idx_src(c):
            return idx_hbm.at[pl.ds(base + c * chunk, chunk)]

        def out_dst(c):
            return out_hbm.at[pl.ds(base + c * chunk, chunk)]

        pltpu.async_copy(idx_src(0), ib[0], isem[0])
        pltpu.async_copy(idx_src(1), ib[1], isem[1])
        pltpu.make_async_copy(idx_src(0), ib[0], isem[0]).wait()
        pltpu.async_copy(table_hbm.at[ib[0]], rb[0], gsem[0])
        pltpu.make_async_copy(idx_src(1), ib[1], isem[1]).wait()
        pltpu.async_copy(idx_src(2), ib[2], isem[2])
        pltpu.async_copy(table_hbm.at[ib[1]], rb[1], gsem[1])

        def body(i, carry):
            def stage(j):
                pltpu.make_async_copy(
                    table_hbm.at[ib[j]], rb[j], gsem[j]).wait()
                pltpu.async_copy(rb[j], out_dst(i), wsem[j])

                @pl.when(i + 2 < nchunk)
                def _():
                    jn = (j + 2) % 3
                    pltpu.make_async_copy(
                        idx_src(i + 2), ib[jn], isem[jn]).wait()

                    @pl.when(i >= 1)
                    def _():
                        pltpu.make_async_copy(
                            rb[jn], out_dst(i - 1), wsem[jn]).wait()

                    pltpu.async_copy(table_hbm.at[ib[jn]], rb[jn], gsem[jn])

                    @pl.when(i + 3 < nchunk)
                    def _():
                        pltpu.async_copy(idx_src(i + 3), ib[j], isem[j])

            @pl.when(i % 3 == 0)
            def _():
                stage(0)

            @pl.when(i % 3 == 1)
            def _():
                stage(1)

            @pl.when(i % 3 == 2)
            def _():
                stage(2)

            return carry

        lax.fori_loop(0, nchunk, body, 0)

        for c in (nchunk - 3, nchunk - 2, nchunk - 1):
            pltpu.make_async_copy(
                rb[c % 3], out_dst(c), wsem[c % 3]).wait()

    return gather_k


_gather = _make_gather(_NROWS_H, _PER_W, _CHUNK, _NCHUNK)

_BM = 256


def _make_mm_body(nblk, has_alias=False):
    if has_alias:

        def _mm_alias_body(x_hbm, w_ref, b_ref, o_in, o_ref,
                           xb0, xb1, sem0, sem1):
            del o_in
            return _make_mm_body(nblk)(
                x_hbm, w_ref, b_ref, o_ref, xb0, xb1, sem0, sem1)

        return _mm_alias_body

    def _mm_body(x_hbm, w_ref, b_ref, o_ref, xb0, xb1, sem0, sem1):
        i = pl.program_id(0)

        def fire(blk, buf, sem):
            for s in range(SEQ):
                pltpu.make_async_copy(
                    x_hbm.at[s, pl.ds(blk * _BM, _BM), :],
                    buf.at[:, pl.ds(s * EMB, EMB)],
                    sem,
                ).start()

        def drain(blk, buf, sem):
            for s in range(SEQ):
                pltpu.make_async_copy(
                    x_hbm.at[s, pl.ds(blk * _BM, _BM), :],
                    buf.at[:, pl.ds(s * EMB, EMB)],
                    sem,
                ).wait()

        @pl.when(i == 0)
        def _():
            fire(0, xb0, sem0)

        @pl.when(i + 1 < nblk)
        def _():

            @pl.when(i % 2 == 0)
            def _():
                fire(i + 1, xb1, sem1)

            @pl.when(i % 2 == 1)
            def _():
                fire(i + 1, xb0, sem0)

        def compute(buf, sem):
            drain(i, buf, sem)
            o_ref[...] = jnp.broadcast_to(
                b_ref[...], o_ref.shape
            ) + lax.dot_general(
                buf[...].astype(jnp.bfloat16),
                w_ref[...],
                (((1,), (1,)), ((), ())),
                preferred_element_type=jnp.float32,
            )

        @pl.when(i % 2 == 0)
        def _():
            compute(xb0, sem0)

        @pl.when(i % 2 == 1)
        def _():
            compute(xb1, sem1)

    return _mm_body


def _matmul(x3, Wb, b2, h, o_prev):
    nblk = _BATCH_H // _BM
    args = [x3, Wb, b2]
    in_specs = [
        pl.BlockSpec(memory_space=pl.ANY),
        pl.BlockSpec((NUM_LABELS, K), lambda i: (0, 0)),
        pl.BlockSpec((1, NUM_LABELS), lambda i: (0, 0)),
    ]
    kwargs = {}
    if o_prev is not None:
        args.append(o_prev)
        in_specs.append(pl.BlockSpec(memory_space=pl.ANY))
        kwargs["input_output_aliases"] = {3: 0}
    return pl.pallas_call(
        _make_mm_body(nblk, has_alias=o_prev is not None),
        grid=(nblk,),
        in_specs=in_specs,
        out_specs=pl.BlockSpec(
            (_BM, NUM_LABELS), lambda i, h=h: (h * nblk + i, 0)
        ),
        out_shape=jax.ShapeDtypeStruct((BATCH, NUM_LABELS), jnp.float32),
        scratch_shapes=[
            pltpu.VMEM((_BM, K), jnp.float32),
            pltpu.VMEM((_BM, K), jnp.float32),
            pltpu.SemaphoreType.DMA,
            pltpu.SemaphoreType.DMA,
        ],
        compiler_params=pltpu.CompilerParams(
            dimension_semantics=("arbitrary",),
        ),
        **kwargs,
    )(*args)


def kernel(words, table, W, b):
    Wb = W.astype(jnp.bfloat16)
    b2 = b.reshape(1, NUM_LABELS)
    xs = []
    for h in range(_H):
        wh = words[h * _BATCH_H:(h + 1) * _BATCH_H]
        idx = wh.T.reshape(-1).astype(jnp.int32)
        rows = _gather(idx, table)
        xs.append(rows.reshape(SEQ, _BATCH_H, EMB))
    out = None
    for h, x3 in enumerate(xs):
        out = _matmul(x3, Wb, b2, h, out)
    return out


# speedup vs baseline: 1.0711x; 1.0711x over previous
import functools

import jax
import jax.numpy as jnp
from jax import lax
from jax.experimental import pallas as pl
from jax.experimental.pallas import tpu as pltpu
from jax.experimental.pallas import tpu_sc as plsc

VOCAB = 100000
EMB = 128
SEQ = 50
BATCH = 4096
NUM_LABELS = 1024
NROWS = BATCH * SEQ
K = SEQ * EMB

_info = plsc.get_sparse_core_info()
_NC, _NS = _info.num_cores, _info.num_subcores
_NW = _NC * _NS
_PER_W = NROWS // _NW
_CHUNK = 200
_NCHUNK = _PER_W // _CHUNK
_NB = 4


def _make_gather():
    mesh = plsc.VectorSubcoreMesh(core_axis_name="c", subcore_axis_name="s")

    @functools.partial(
        pl.kernel,
        mesh=mesh,
        out_type=jax.ShapeDtypeStruct((NROWS, EMB), jnp.float32),
        scratch_types=(
            [pltpu.VMEM((_CHUNK,), jnp.int32)] * _NB
            + [pltpu.VMEM((_CHUNK, EMB), jnp.float32)] * _NB
            + [pltpu.SemaphoreType.DMA] * (3 * _NB)
        ),
    )
    def gather_k(idx_hbm, table_hbm, out_hbm, *bufs):
        ib = bufs[0:_NB]
        rb = bufs[_NB:2 * _NB]
        isem = bufs[2 * _NB:3 * _NB]
        gsem = bufs[3 * _NB:4 * _NB]
        wsem = bufs[4 * _NB:5 * _NB]
        wid = lax.axis_index("s") * _NC + lax.axis_index("c")
        base = wid * _PER_W

        def idx_src(c):
            return idx_hbm.at[pl.ds(base + c * _CHUNK, _CHUNK)]

        def out_dst(c):
            return out_hbm.at[pl.ds(base + c * _CHUNK, _CHUNK)]

        for c in range(_NB):
            pltpu.async_copy(idx_src(c), ib[c], isem[c])
        for c in range(_NB - 1):
            pltpu.make_async_copy(idx_src(c), ib[c], isem[c]).wait()
            pltpu.async_copy(table_hbm.at[ib[c]], rb[c], gsem[c])

        def body(i, carry):
            def stage(j):
                pltpu.make_async_copy(
                    table_hbm.at[ib[j]], rb[j], gsem[j]).wait()
                pltpu.async_copy(rb[j], out_dst(i), wsem[j])

                @pl.when(i + _NB - 1 < _NCHUNK)
                def _():
                    jn = (j + _NB - 1) % _NB
                    pltpu.make_async_copy(
                        idx_src(i + _NB - 1), ib[jn], isem[jn]).wait()

                    @pl.when(i >= 1)
                    def _():
                        pltpu.make_async_copy(
                            rb[jn], out_dst(i - 1), wsem[jn]).wait()

                    pltpu.async_copy(table_hbm.at[ib[jn]], rb[jn], gsem[jn])

                    @pl.when(i + _NB < _NCHUNK)
                    def _():
                        pltpu.async_copy(idx_src(i + _NB), ib[j], isem[j])

            for j in range(_NB):

                @pl.when(i % _NB == j)
                def _(j=j):
                    stage(j)

            return carry

        lax.fori_loop(0, _NCHUNK, body, 0)

        for c in range(_NCHUNK - _NB, _NCHUNK):
            pltpu.make_async_copy(
                rb[c % _NB], out_dst(c), wsem[c % _NB]).wait()

    return gather_k


_gather = _make_gather()

_BM = 256
_NBLK = BATCH // _BM


def _mm_body(x_hbm, w_ref, b_ref, o_ref, xb0, xb1, sem0, sem1):
    i = pl.program_id(0)
    bufs = (xb0, xb1)
    sems = (sem0, sem1)

    def fire(blk, buf, sem):
        for s in range(SEQ):
            pltpu.make_async_copy(
                x_hbm.at[s, pl.ds(blk * _BM, _BM), :],
                buf.at[:, pl.ds(s * EMB, EMB)],
                sem,
            ).start()

    def drain(blk, buf, sem):
        for s in range(SEQ):
            pltpu.make_async_copy(
                x_hbm.at[s, pl.ds(blk * _BM, _BM), :],
                buf.at[:, pl.ds(s * EMB, EMB)],
                sem,
            ).wait()

    @pl.when(i == 0)
    def _():
        fire(0, xb0, sem0)

    @pl.when(i + 1 < _NBLK)
    def _():

        @pl.when(i % 2 == 0)
        def _():
            fire(i + 1, xb1, sem1)

        @pl.when(i % 2 == 1)
        def _():
            fire(i + 1, xb0, sem0)

    def compute(buf, sem):
        drain(i, buf, sem)
        o_ref[...] = jnp.broadcast_to(b_ref[...], o_ref.shape) + lax.dot_general(
            buf[...].astype(jnp.bfloat16),
            w_ref[...],
            (((1,), (1,)), ((), ())),
            preferred_element_type=jnp.float32,
        )

    @pl.when(i % 2 == 0)
    def _():
        compute(xb0, sem0)

    @pl.when(i % 2 == 1)
    def _():
        compute(xb1, sem1)


def _matmul(x3, Wb, b2):
    return pl.pallas_call(
        _mm_body,
        grid=(_NBLK,),
        in_specs=[
            pl.BlockSpec(memory_space=pl.ANY),
            pl.BlockSpec((NUM_LABELS, K), lambda i: (0, 0)),
            pl.BlockSpec((1, NUM_LABELS), lambda i: (0, 0)),
        ],
        out_specs=pl.BlockSpec((_BM, NUM_LABELS), lambda i: (i, 0)),
        out_shape=jax.ShapeDtypeStruct((BATCH, NUM_LABELS), jnp.float32),
        scratch_shapes=[
            pltpu.VMEM((_BM, K), jnp.float32),
            pltpu.VMEM((_BM, K), jnp.float32),
            pltpu.SemaphoreType.DMA,
            pltpu.SemaphoreType.DMA,
        ],
        compiler_params=pltpu.CompilerParams(
            dimension_semantics=("arbitrary",),
        ),
    )(x3, Wb, b2)


def kernel(words, table, W, b):
    idx = words.T.reshape(-1).astype(jnp.int32)
    rows = _gather(idx, table)
    x3 = rows.reshape(SEQ, BATCH, EMB)
    Wb = W.astype(jnp.bfloat16)
    return _matmul(x3, Wb, b.reshape(1, NUM_LABELS))


# speedup vs baseline: 1.1031x; 1.0299x over previous
import functools

import jax
import jax.numpy as jnp
from jax import lax
from jax.experimental import pallas as pl
from jax.experimental.pallas import tpu as pltpu
from jax.experimental.pallas import tpu_sc as plsc

VOCAB = 100000
EMB = 128
SEQ = 50
BATCH = 4096
NUM_LABELS = 1024
K = SEQ * EMB

_info = plsc.get_sparse_core_info()
_NC, _NS = _info.num_cores, _info.num_subcores
_NW = _NC * _NS

_H = 2
_BATCH_H = BATCH // _H
_NROWS_H = _BATCH_H * SEQ
_PER_W = _NROWS_H // _NW
_CHUNK = 320
_NCHUNK = _PER_W // _CHUNK


def _make_gather(nrows, per_w, chunk, nchunk):
    mesh = plsc.VectorSubcoreMesh(core_axis_name="c", subcore_axis_name="s")

    @functools.partial(
        pl.kernel,
        mesh=mesh,
        out_type=jax.ShapeDtypeStruct((nrows, EMB), jnp.float32),
        scratch_types=[
            pltpu.VMEM((chunk,), jnp.int32),
            pltpu.VMEM((chunk,), jnp.int32),
            pltpu.VMEM((chunk,), jnp.int32),
            pltpu.VMEM((chunk, EMB), jnp.float32),
            pltpu.VMEM((chunk, EMB), jnp.float32),
            pltpu.VMEM((chunk, EMB), jnp.float32),
            pltpu.SemaphoreType.DMA,
            pltpu.SemaphoreType.DMA,
            pltpu.SemaphoreType.DMA,
            pltpu.SemaphoreType.DMA,
            pltpu.SemaphoreType.DMA,
            pltpu.SemaphoreType.DMA,
            pltpu.SemaphoreType.DMA,
            pltpu.SemaphoreType.DMA,
            pltpu.SemaphoreType.DMA,
        ],
    )
    def gather_k(idx_hbm, table_hbm, out_hbm,
                 ib0, ib1, ib2, rb0, rb1, rb2,
                 is0, is1, is2, gs0, gs1, gs2, ws0, ws1, ws2):
        wid = lax.axis_index("s") * _NC + lax.axis_index("c")
        base = wid * per_w
        ib = (ib0, ib1, ib2)
        rb = (rb0, rb1, rb2)
        isem = (is0, is1, is2)
        gsem = (gs0, gs1, gs2)
        wsem = (ws0, ws1, ws2)

        def idx_src(c):
            return idx_hbm.at[pl.ds(base + c * chunk, chunk)]

        def out_dst(c):
            return out_hbm.at[pl.ds(base + c * chunk, chunk)]

        pltpu.async_copy(idx_src(0), ib[0], isem[0])
        pltpu.async_copy(idx_src(1), ib[1], isem[1])
        pltpu.make_async_copy(idx_src(0), ib[0], isem[0]).wait()
        pltpu.async_copy(table_hbm.at[ib[0]], rb[0], gsem[0])
        pltpu.make_async_copy(idx_src(1), ib[1], isem[1]).wait()
        pltpu.async_copy(idx_src(2), ib[2], isem[2])
        pltpu.async_copy(table_hbm.at[ib[1]], rb[1], gsem[1])

        def body(i, carry):
            def stage(j):
                pltpu.make_async_copy(
                    table_hbm.at[ib[j]], rb[j], gsem[j]).wait()
                pltpu.async_copy(rb[j], out_dst(i), wsem[j])

                @pl.when(i + 2 < nchunk)
                def _():
                    jn = (j + 2) % 3
                    pltpu.make_async_copy(
                        idx_src(i + 2), ib[jn], isem[jn]).wait()

                    @pl.when(i >= 1)
                    def _():
                        pltpu.make_async_copy(
                            rb[jn], out_dst(i - 1), wsem[jn]).wait()

                    pltpu.async_copy(table_hbm.at[ib[jn]], rb[jn], gsem[jn])

                    @pl.when(i + 3 < nchunk)
                    def _():
                        pltpu.async_copy(idx_src(i + 3), ib[j], isem[j])

            @pl.when(i % 3 == 0)
            def _():
                stage(0)

            @pl.when(i % 3 == 1)
            def _():
                stage(1)

            @pl.when(i % 3 == 2)
            def _():
                stage(2)

            return carry

        lax.fori_loop(0, nchunk, body, 0)

        for c in (nchunk - 3, nchunk - 2, nchunk - 1):
            pltpu.make_async_copy(
                rb[c % 3], out_dst(c), wsem[c % 3]).wait()

    return gather_k


_gather = _make_gather(_NROWS_H, _PER_W, _CHUNK, _NCHUNK)

_BM = 256


def _make_mm_body(nblk, has_alias=False):
    if has_alias:

        def _mm_alias_body(x_hbm, w_ref, b_ref, o_in, o_ref,
                           xb0, xb1, sem0, sem1):
            del o_in
            return _make_mm_body(nblk)(
                x_hbm, w_ref, b_ref, o_ref, xb0, xb1, sem0, sem1)

        return _mm_alias_body

    def _mm_body(x_hbm, w_ref, b_ref, o_ref, xb0, xb1, sem0, sem1):
        i = pl.program_id(0)

        def fire(blk, buf, sem):
            for s in range(SEQ):
                pltpu.make_async_copy(
                    x_hbm.at[s, pl.ds(blk * _BM, _BM), :],
                    buf.at[:, pl.ds(s * EMB, EMB)],
                    sem,
                ).start()

        def drain(blk, buf, sem):
            for s in range(SEQ):
                pltpu.make_async_copy(
                    x_hbm.at[s, pl.ds(blk * _BM, _BM), :],
                    buf.at[:, pl.ds(s * EMB, EMB)],
                    sem,
                ).wait()

        @pl.when(i == 0)
        def _():
            fire(0, xb0, sem0)

        @pl.when(i + 1 < nblk)
        def _():

            @pl.when(i % 2 == 0)
            def _():
                fire(i + 1, xb1, sem1)

            @pl.when(i % 2 == 1)
            def _():
                fire(i + 1, xb0, sem0)

        def compute(buf, sem):
            drain(i, buf, sem)
            o_ref[...] = jnp.broadcast_to(
                b_ref[...], o_ref.shape
            ) + lax.dot_general(
                buf[...].astype(jnp.bfloat16),
                w_ref[...],
                (((1,), (1,)), ((), ())),
                preferred_element_type=jnp.float32,
            )

        @pl.when(i % 2 == 0)
        def _():
            compute(xb0, sem0)

        @pl.when(i % 2 == 1)
        def _():
            compute(xb1, sem1)

    return _mm_body


def _matmul(x3, Wb, b2, h, o_prev):
    nblk = _BATCH_H // _BM
    args = [x3, Wb, b2]
    in_specs = [
        pl.BlockSpec(memory_space=pl.ANY),
        pl.BlockSpec((NUM_LABELS, K), lambda i: (0, 0)),
        pl.BlockSpec((1, NUM_LABELS), lambda i: (0, 0)),
    ]
    kwargs = {}
    if o_prev is not None:
        args.append(o_prev)
        in_specs.append(pl.BlockSpec(memory_space=pl.ANY))
        kwargs["input_output_aliases"] = {3: 0}
    return pl.pallas_call(
        _make_mm_body(nblk, has_alias=o_prev is not None),
        grid=(nblk,),
        in_specs=in_specs,
        out_specs=pl.BlockSpec(
            (_BM, NUM_LABELS), lambda i, h=h: (h * nblk + i, 0)
        ),
        out_shape=jax.ShapeDtypeStruct((BATCH, NUM_LABELS), jnp.float32),
        scratch_shapes=[
            pltpu.VMEM((_BM, K), jnp.float32),
            pltpu.VMEM((_BM, K), jnp.float32),
            pltpu.SemaphoreType.DMA,
            pltpu.SemaphoreType.DMA,
        ],
        compiler_params=pltpu.CompilerParams(
            dimension_semantics=("arbitrary",),
        ),
        **kwargs,
    )(*args)


def kernel(words, table, W, b):
    Wb = W.astype(jnp.bfloat16)
    b2 = b.reshape(1, NUM_LABELS)
    xs = []
    for h in range(_H):
        wh = words[h * _BATCH_H:(h + 1) * _BATCH_H]
        idx = wh.T.reshape(-1).astype(jnp.int32)
        rows = _gather(idx, table)
        xs.append(rows.reshape(SEQ, _BATCH_H, EMB))
    out = None
    for h, x3 in enumerate(xs):
        out = _matmul(x3, Wb, b2, h, out)
    return out
